# baseline (device time: 11807 ns/iter reference)
import jax
import jax.numpy as jnp
from jax import lax
from jax.experimental import pallas as pl
from jax.experimental.pallas import tpu as pltpu

CHUNKS = (16, 112, 112, 16)
K = len(CHUNKS)
OFFS = tuple(sum(CHUNKS[:i]) for i in range(K))


def kernel(x):
    x = pltpu.with_memory_space_constraint(x, pltpu.MemorySpace.HBM)
    _, M, N2 = x.shape
    N = N2 // 2
    H = M // 2
    assert sum(CHUNKS) == H

    def body(x_hbm, out_ref, rows_my, loc_other, xsend, xrecv, yrecv,
             r_sems, o_sem, xs_sems, xr_sems, ys_sems, yr_sems):
        my_x = lax.axis_index("x")
        my_y = lax.axis_index("y")
        xpeer = (1 - my_x, my_y)
        ypeer = (my_x, 1 - my_y)
        row0 = my_y * H
        other0 = (1 - my_y) * H
        pcol0 = (1 - my_x) * N
        mcol0 = my_x * N

        barrier_sem = pltpu.get_barrier_semaphore()
        for nbr in (xpeer, ypeer):
            pl.semaphore_signal(
                barrier_sem, inc=1, device_id=nbr,
                device_id_type=pl.DeviceIdType.MESH,
            )

        dma_r = []
        for k in range(K):
            d = pltpu.make_async_copy(
                x_hbm.at[0, pl.ds(row0 + OFFS[k], CHUNKS[k]), :],
                rows_my.at[pl.ds(OFFS[k], CHUNKS[k])],
                r_sems.at[k],
            )
            d.start()
            dma_r.append(d)
        dma_o = pltpu.make_async_copy(
            x_hbm.at[0, pl.ds(other0, H), pl.ds(mcol0, N)], loc_other, o_sem
        )
        dma_o.start()

        dma_r[0].wait()
        xsend[pl.ds(0, CHUNKS[0])] = rows_my[
            pl.ds(0, CHUNKS[0]), pl.ds(pcol0, N)
        ].astype(jnp.bfloat16)

        pl.semaphore_wait(barrier_sem, 2)

        xrd = []
        for k in range(K):
            r = pltpu.make_async_remote_copy(
                src_ref=xsend.at[pl.ds(OFFS[k], CHUNKS[k])],
                dst_ref=xrecv.at[pl.ds(OFFS[k], CHUNKS[k])],
                send_sem=xs_sems.at[k],
                recv_sem=xr_sems.at[k],
                device_id=xpeer,
                device_id_type=pl.DeviceIdType.MESH,
            )
            r.start()
            xrd.append(r)
            if k + 1 < K:
                dma_r[k + 1].wait()
                xsend[pl.ds(OFFS[k + 1], CHUNKS[k + 1])] = rows_my[
                    pl.ds(OFFS[k + 1], CHUNKS[k + 1]), pl.ds(pcol0, N)
                ].astype(jnp.bfloat16)

        yrd = []
        for k in range(K):
            xrd[k].wait_recv()
            r = pltpu.make_async_remote_copy(
                src_ref=xrecv.at[pl.ds(OFFS[k], CHUNKS[k])],
                dst_ref=yrecv.at[pl.ds(OFFS[k], CHUNKS[k])],
                send_sem=ys_sems.at[k],
                recv_sem=yr_sems.at[k],
                device_id=ypeer,
                device_id_type=pl.DeviceIdType.MESH,
            )
            r.start()
            yrd.append(r)

        out_ref[pl.ds(row0, H), :] = (
            rows_my[:, pl.ds(mcol0, N)].astype(jnp.bfloat16) + xrecv[...]
        )

        dma_o.wait()
        for k in range(K):
            yrd[k].wait_recv()
            out_ref[pl.ds(other0 + OFFS[k], CHUNKS[k]), :] = (
                loc_other[pl.ds(OFFS[k], CHUNKS[k])].astype(jnp.bfloat16)
                + yrecv[pl.ds(OFFS[k], CHUNKS[k])]
            )

        for k in range(K):
            xrd[k].wait_send()
            yrd[k].wait_send()

    return pl.pallas_call(
        body,
        out_shape=jax.ShapeDtypeStruct((M, N), jnp.bfloat16),
        in_specs=[pl.BlockSpec(memory_space=pltpu.MemorySpace.HBM)],
        out_specs=pl.BlockSpec(memory_space=pltpu.VMEM),
        scratch_shapes=[
            pltpu.VMEM((H, N2), jnp.float32),
            pltpu.VMEM((H, N), jnp.float32),
            pltpu.VMEM((H, N), jnp.bfloat16),
            pltpu.VMEM((H, N), jnp.bfloat16),
            pltpu.VMEM((H, N), jnp.bfloat16),
            pltpu.SemaphoreType.DMA((K,)),
            pltpu.SemaphoreType.DMA,
            pltpu.SemaphoreType.DMA((K,)),
            pltpu.SemaphoreType.DMA((K,)),
            pltpu.SemaphoreType.DMA((K,)),
            pltpu.SemaphoreType.DMA((K,)),
        ],
        compiler_params=pltpu.CompilerParams(collective_id=0),
    )(x)


# device time: 11297 ns/iter; 1.0451x vs baseline; 1.0451x over previous
import jax
import jax.numpy as jnp
from jax import lax
from jax.experimental import pallas as pl
from jax.experimental.pallas import tpu as pltpu

CHUNKS = (64, 64, 64, 64)
K = len(CHUNKS)
OFFS = tuple(sum(CHUNKS[:i]) for i in range(K))


def kernel(x):
    x = pltpu.with_memory_space_constraint(x, pltpu.MemorySpace.HBM)
    _, M, N2 = x.shape
    N = N2 // 2
    H = M // 2
    assert sum(CHUNKS) == H

    def body(x_hbm, out_ref, xstage, loc_my, loc_other, xsend, xrecv, yrecv,
             s_sems, dma_sems, xs_sems, xr_sems, ys_sems, yr_sems):
        my_x = lax.axis_index("x")
        my_y = lax.axis_index("y")
        xpeer = (1 - my_x, my_y)
        ypeer = (my_x, 1 - my_y)
        row0 = my_y * H
        other0 = (1 - my_y) * H
        pcol0 = (1 - my_x) * N
        mcol0 = my_x * N

        barrier_sem = pltpu.get_barrier_semaphore()
        for nbr in (xpeer, ypeer):
            pl.semaphore_signal(
                barrier_sem, inc=1, device_id=nbr,
                device_id_type=pl.DeviceIdType.MESH,
            )

        dma_s = []
        for k in range(K):
            d = pltpu.make_async_copy(
                x_hbm.at[0, pl.ds(row0 + OFFS[k], CHUNKS[k]), pl.ds(pcol0, N)],
                xstage.at[pl.ds(OFFS[k], CHUNKS[k])],
                s_sems.at[k],
            )
            d.start()
            dma_s.append(d)
        dma_m = pltpu.make_async_copy(
            x_hbm.at[0, pl.ds(row0, H), pl.ds(mcol0, N)], loc_my,
            dma_sems.at[0],
        )
        dma_m.start()
        dma_o = pltpu.make_async_copy(
            x_hbm.at[0, pl.ds(other0, H), pl.ds(mcol0, N)], loc_other,
            dma_sems.at[1],
        )
        dma_o.start()

        dma_s[0].wait()
        xsend[pl.ds(0, CHUNKS[0])] = xstage[pl.ds(0, CHUNKS[0])].astype(
            jnp.bfloat16
        )

        pl.semaphore_wait(barrier_sem, 2)

        xrd = []
        for k in range(K):
            r = pltpu.make_async_remote_copy(
                src_ref=xsend.at[pl.ds(OFFS[k], CHUNKS[k])],
                dst_ref=xrecv.at[pl.ds(OFFS[k], CHUNKS[k])],
                send_sem=xs_sems.at[k],
                recv_sem=xr_sems.at[k],
                device_id=xpeer,
                device_id_type=pl.DeviceIdType.MESH,
            )
            r.start()
            xrd.append(r)
            if k + 1 < K:
                dma_s[k + 1].wait()
                xsend[pl.ds(OFFS[k + 1], CHUNKS[k + 1])] = xstage[
                    pl.ds(OFFS[k + 1], CHUNKS[k + 1])
                ].astype(jnp.bfloat16)

        yrd = []
        for k in range(K):
            xrd[k].wait_recv()
            r = pltpu.make_async_remote_copy(
                src_ref=xrecv.at[pl.ds(OFFS[k], CHUNKS[k])],
                dst_ref=yrecv.at[pl.ds(OFFS[k], CHUNKS[k])],
                send_sem=ys_sems.at[k],
                recv_sem=yr_sems.at[k],
                device_id=ypeer,
                device_id_type=pl.DeviceIdType.MESH,
            )
            r.start()
            yrd.append(r)

        dma_m.wait()
        out_ref[pl.ds(row0, H), :] = (
            loc_my[...].astype(jnp.bfloat16) + xrecv[...]
        )

        dma_o.wait()
        for k in range(K):
            yrd[k].wait_recv()
            out_ref[pl.ds(other0 + OFFS[k], CHUNKS[k]), :] = (
                loc_other[pl.ds(OFFS[k], CHUNKS[k])].astype(jnp.bfloat16)
                + yrecv[pl.ds(OFFS[k], CHUNKS[k])]
            )

        for k in range(K):
            xrd[k].wait_send()
            yrd[k].wait_send()

    return pl.pallas_call(
        body,
        out_shape=jax.ShapeDtypeStruct((M, N), jnp.bfloat16),
        in_specs=[pl.BlockSpec(memory_space=pltpu.MemorySpace.HBM)],
        out_specs=pl.BlockSpec(memory_space=pltpu.VMEM),
        scratch_shapes=[
            pltpu.VMEM((H, N), jnp.float32),
            pltpu.VMEM((H, N), jnp.float32),
            pltpu.VMEM((H, N), jnp.float32),
            pltpu.VMEM((H, N), jnp.bfloat16),
            pltpu.VMEM((H, N), jnp.bfloat16),
            pltpu.VMEM((H, N), jnp.bfloat16),
            pltpu.SemaphoreType.DMA((K,)),
            pltpu.SemaphoreType.DMA((2,)),
            pltpu.SemaphoreType.DMA((K,)),
            pltpu.SemaphoreType.DMA((K,)),
            pltpu.SemaphoreType.DMA((K,)),
            pltpu.SemaphoreType.DMA((K,)),
        ],
        compiler_params=pltpu.CompilerParams(collective_id=0),
    )(x)
